# Initial kernel scaffold; baseline (speedup 1.0000x reference)
import jax
import jax.numpy as jnp
from jax.experimental import pallas as pl


def kernel(x0, x1, x2, edge_index0, edge_index1, edge_index2, edge_index_next, W1, b1, W2, b2, Wa, va, We1, be1, We2, be2):
    raise NotImplementedError("write your pallas kernel here")


# speedup vs baseline: 11.0663x; 11.0663x over previous
import functools

import jax
import jax.numpy as jnp
from jax import lax
from jax.experimental import pallas as pl
from jax.experimental.pallas import tpu as pltpu
from jax.experimental.pallas import tpu_sc as plsc

N = 10000
NP = 10240
E = 320000
D = 128
T = 3
NC = 2
NS = 16
NW = NC * NS
EPW = E // NW
CH = 80
NCHUNK = EPW // CH
RPT = NP // NS
BN = 640
NB = NP // BN

@functools.lru_cache(maxsize=None)
def _mesh():
    return plsc.VectorSubcoreMesh(
        core_axis_name="c", subcore_axis_name="s", num_cores=NC, num_subcores=NS)


def _wid():
    c = lax.axis_index("c")
    s = lax.axis_index("s")
    return c, s, c * NS + s


def _al8(x):
    return pl.multiple_of(x, 8)


@functools.lru_cache(maxsize=None)
def _build_deg_kernel():
  return functools.partial(
      pl.kernel,
      out_type=jax.ShapeDtypeStruct((NW * T * NP,), jnp.float32),
      mesh=_mesh(),
      compiler_params=pltpu.CompilerParams(needs_layout_passes=False),
      scratch_types=[
          pltpu.VMEM((NP,), jnp.float32),
          pltpu.VMEM((EPW,), jnp.int32),
      ],
  )(_deg_body)


def _deg_body(dst_hbm, out_hbm, hist, dall):
    c_id, s_id, w = _wid()
    base = w * EPW
    z16 = jnp.zeros((16,), jnp.float32)
    o16 = jnp.ones((16,), jnp.float32)

    for t in range(T):
        pltpu.sync_copy(dst_hbm.at[pl.ds(_al8(t * E + base), EPW)], dall)

        def zero(i, _):
            hist[pl.ds(i * 16, 16)] = z16
            return 0

        lax.fori_loop(0, NP // 16, zero, 0)

        def scat(m, _):
            v = dall[pl.ds(m * 16, 16)]
            plsc.addupdate_scatter(hist, [v], o16)
            return 0

        lax.fori_loop(0, EPW // 16, scat, 0)
        pltpu.sync_copy(hist, out_hbm.at[pl.ds(_al8((w * T + t) * NP), NP)])


@functools.lru_cache(maxsize=None)
def _build_segsum_kernel():
  return functools.partial(
      pl.kernel,
      out_type=jax.ShapeDtypeStruct((NC, T, NP, D), jnp.float32),
      mesh=_mesh(),
      compiler_params=pltpu.CompilerParams(needs_layout_passes=False),
      scratch_types=[
          pltpu.VMEM((32, D), jnp.float32),
          pltpu.VMEM((EPW,), jnp.int32),
          pltpu.VMEM((EPW,), jnp.int32),
          pltpu.VMEM((CH,), jnp.int32),
          pltpu.VMEM((CH,), jnp.int32),
          pltpu.VMEM((CH,), jnp.int32),
          pltpu.VMEM((CH,), jnp.int32),
          pltpu.VMEM((CH, D), jnp.float32),
          pltpu.VMEM((CH, D), jnp.float32),
          pltpu.VMEM_SHARED((NP, D), jnp.float32),
          pltpu.SemaphoreType.DMA,
          pltpu.SemaphoreType.DMA,
          pltpu.SemaphoreType.DMA,
          pltpu.SemaphoreType.DMA,
      ],
  )(_segsum_body)


def _segsum_body(src_hbm, dst_hbm, xs_hbm, out_hbm, zbuf, sall, dall,
                 sia, dia, sib, dib, rowa, rowb, shared, sga, sgb, ssa, ssb):
    c_id, s_id, w = _wid()
    base = w * EPW
    z16 = jnp.zeros((16,), jnp.float32)

    def init_rows(i, _):
        for cc in range(D // 16):
            zbuf[i, pl.ds(cc * 16, 16)] = z16
        return 0

    lax.fori_loop(0, 32, init_rows, 0)

    for t in range(T):
        pltpu.sync_copy(src_hbm.at[pl.ds(_al8(t * E + base), EPW)], sall)
        pltpu.sync_copy(dst_hbm.at[pl.ds(_al8(t * E + base), EPW)], dall)
        for z in range(RPT // 32):
            pltpu.sync_copy(
                zbuf, shared.at[pl.ds(_al8(s_id * RPT + z * 32), 32), :])
        plsc.subcore_barrier()

        def prep(i, si, di):
            for u in range(CH // 16):
                v = sall[pl.ds(i * CH + u * 16, 16)]
                si[pl.ds(u * 16, 16)] = v + t * NP if t else v
                di[pl.ds(u * 16, 16)] = dall[pl.ds(i * CH + u * 16, 16)]

        prep(0, sia, dia)
        pltpu.async_copy(xs_hbm.at[sia], rowa, sga)

        def pair(m, _):
            a = 2 * m

            @pl.when(m > 0)
            def _():
                pltpu.make_async_copy(rowb, shared.at[dib], ssb).wait()

            prep(a + 1, sib, dib)
            pltpu.make_async_copy(xs_hbm.at[sia], rowa, sga).wait()
            pltpu.async_copy(xs_hbm.at[sib], rowb, sgb)
            pltpu.async_copy(rowa, shared.at[dia], ssa, add=True)
            pltpu.make_async_copy(rowa, shared.at[dia], ssa).wait()
            prep(a + 2, sia, dia)
            pltpu.make_async_copy(xs_hbm.at[sib], rowb, sgb).wait()
            pltpu.async_copy(xs_hbm.at[sia], rowa, sga)
            pltpu.async_copy(rowb, shared.at[dib], ssb, add=True)
            return 0

        lax.fori_loop(0, NCHUNK // 2, pair, 0)
        pltpu.make_async_copy(rowb, shared.at[dib], ssb).wait()
        pltpu.make_async_copy(xs_hbm.at[sia], rowa, sga).wait()
        pltpu.sync_copy(rowa, shared.at[dia], add=True)
        plsc.subcore_barrier()
        pltpu.sync_copy(
            shared.at[pl.ds(_al8(s_id * RPT), RPT), :],
            out_hbm.at[c_id, t, pl.ds(_al8(s_id * RPT), RPT), :])


@functools.lru_cache(maxsize=None)
def _build_edge_kernel():
  return functools.partial(
      pl.kernel,
      out_type=jax.ShapeDtypeStruct((E,), jnp.float32),
      mesh=_mesh(),
      compiler_params=pltpu.CompilerParams(needs_layout_passes=False),
      scratch_types=[
          pltpu.VMEM((EPW,), jnp.int32),
          pltpu.VMEM((EPW,), jnp.int32),
          pltpu.VMEM((CH,), jnp.int32),
          pltpu.VMEM((CH,), jnp.int32),
          pltpu.VMEM((CH,), jnp.int32),
          pltpu.VMEM((CH,), jnp.int32),
          pltpu.VMEM((CH, D), jnp.float32),
          pltpu.VMEM((CH, D), jnp.float32),
          pltpu.VMEM((CH, D), jnp.float32),
          pltpu.VMEM((CH, D), jnp.float32),
          pltpu.VMEM((256,), jnp.float32),
          pltpu.VMEM((CH,), jnp.float32),
          pltpu.VMEM((CH,), jnp.float32),
          pltpu.VMEM((D,), jnp.float32),
          pltpu.VMEM((16,), jnp.float32),
          pltpu.SemaphoreType.DMA,
          pltpu.SemaphoreType.DMA,
          pltpu.SemaphoreType.DMA,
          pltpu.SemaphoreType.DMA,
          pltpu.SemaphoreType.DMA,
          pltpu.SemaphoreType.DMA,
      ],
  )(_edge_body)


def _edge_body(p_hbm, q_hbm, src_hbm, dst_hbm, w2_hbm, be2_hbm, out_hbm,
               sall, dall, sia, dia, sib, dib, prowa, qrowa, prowb, qrowb,
               mat, outba, outbb, w2v, be2v, spa, sqa, spb, sqb, soa, sob):
    c_id, s_id, w = _wid()
    base = w * EPW
    pltpu.sync_copy(w2_hbm, w2v)
    pltpu.sync_copy(be2_hbm, be2v)
    pltpu.sync_copy(src_hbm.at[pl.ds(_al8(base), EPW)], sall)
    pltpu.sync_copy(dst_hbm.at[pl.ds(_al8(base), EPW)], dall)
    rows16 = lax.iota(jnp.int32, 16)

    def prep(i, si, di):
        for u in range(CH // 16):
            si[pl.ds(u * 16, 16)] = sall[pl.ds(i * CH + u * 16, 16)]
            di[pl.ds(u * 16, 16)] = dall[pl.ds(i * CH + u * 16, 16)]

    def fire(si, di, prow, qrow, sp, sq):
        pltpu.async_copy(p_hbm.at[si], prow, sp)
        pltpu.async_copy(q_hbm.at[di], qrow, sq)

    def wait_g(si, di, prow, qrow, sp, sq):
        pltpu.make_async_copy(p_hbm.at[si], prow, sp).wait()
        pltpu.make_async_copy(q_hbm.at[di], qrow, sq).wait()

    def compute(prow, qrow, outb):
        w2c = [w2v[pl.ds(cc * 16, 16)] for cc in range(D // 16)]
        be2vec = be2v[...]

        def group(g, _):
            for j in range(16):
                r = g * 16 + j
                acc = jnp.zeros((16,), jnp.float32)
                for cc in range(D // 16):
                    p = prow[r, pl.ds(cc * 16, 16)]
                    q = qrow[r, pl.ds(cc * 16, 16)]
                    rl = _bf16_rtne(jnp.maximum(p + q, 0.0))
                    acc = acc + rl * w2c[cc]
                mat[pl.ds(j * 16, 16)] = acc
            colsum = be2vec
            for l in range(16):
                colsum = colsum + plsc.load_gather(mat, [rows16 * 16 + l])
            outb[pl.ds(g * 16, 16)] = colsum
            return 0

        lax.fori_loop(0, CH // 16, group, 0)

    prep(0, sia, dia)
    fire(sia, dia, prowa, qrowa, spa, sqa)

    def pair(m, _):
        a = 2 * m

        @pl.when(m > 0)
        def _():
            pltpu.make_async_copy(
                outbb, out_hbm.at[pl.ds(_al8(base + (a - 1) * CH), CH)],
                sob).wait()

        prep(a + 1, sib, dib)
        wait_g(sia, dia, prowa, qrowa, spa, sqa)
        fire(sib, dib, prowb, qrowb, spb, sqb)

        @pl.when(m > 0)
        def _():
            pltpu.make_async_copy(
                outba, out_hbm.at[pl.ds(_al8(base + (a - 2) * CH), CH)],
                soa).wait()

        compute(prowa, qrowa, outba)
        pltpu.async_copy(outba, out_hbm.at[pl.ds(_al8(base + a * CH), CH)], soa)
        prep(a + 2, sia, dia)
        wait_g(sib, dib, prowb, qrowb, spb, sqb)
        fire(sia, dia, prowa, qrowa, spa, sqa)
        compute(prowb, qrowb, outbb)
        pltpu.async_copy(
            outbb, out_hbm.at[pl.ds(_al8(base + (a + 1) * CH), CH)], sob)
        return 0

    lax.fori_loop(0, NCHUNK // 2, pair, 0)
    pltpu.make_async_copy(
        outbb, out_hbm.at[pl.ds(_al8(base + (NCHUNK - 2) * CH), CH)],
        sob).wait()
    pltpu.make_async_copy(
        outba, out_hbm.at[pl.ds(_al8(base + (NCHUNK - 3) * CH), CH)],
        soa).wait()
    wait_g(sia, dia, prowa, qrowa, spa, sqa)
    compute(prowa, qrowa, outba)
    pltpu.sync_copy(outba, out_hbm.at[pl.ds(_al8(base + (NCHUNK - 1) * CH), CH)])


def _rsqrt_exact(x):
    r = lax.rsqrt(x)
    r = r * (1.5 - 0.5 * x * r * r)
    r = r * (1.5 - 0.5 * x * r * r)
    return r


_PREC = lax.Precision.HIGHEST


def _bf16_rtne(x):
    u = lax.bitcast_convert_type(x, jnp.int32)
    b = u + jnp.int32(0x7FFF) + (lax.shift_right_logical(u, 16) & jnp.int32(1))
    return lax.bitcast_convert_type(b & jnp.int32(-65536), jnp.float32)


def _b_body(x_ref, degp_ref, w1_ref, xs_ref, norm_ref):
    degp = degp_ref[...]
    w1 = w1_ref[...]
    for t in range(T):
        deg = jnp.sum(degp[:, t], axis=0) + 1.0
        nrm = _rsqrt_exact(deg)
        xw = jnp.dot(_bf16_rtne(x_ref[t]), _bf16_rtne(w1),
                     preferred_element_type=jnp.float32)
        xs_ref[t, :, :] = xw * nrm[:, None]
        norm_ref[t, :] = nrm


def _d_body(agg_ref, xs_ref, norm_ref, b1_ref, w2_ref, xs2_ref):
    w2 = w2_ref[...]
    b1 = b1_ref[...]
    for t in range(T):
        a = agg_ref[0, t] + agg_ref[1, t] + xs_ref[t]
        h1 = jnp.maximum(a * norm_ref[t][:, None] + b1, 0.0)
        xw = jnp.dot(_bf16_rtne(h1), _bf16_rtne(w2),
                     preferred_element_type=jnp.float32)
        xs2_ref[t, :, :] = xw * norm_ref[t][:, None]


def _f_body(agg2_ref, xs2_ref, norm_ref, b2_ref, wa_ref, va_ref, wtop_ref,
            wbot_ref, be1_ref, p_ref, q_ref):
    wa = wa_ref[...]
    va = va_ref[...].reshape(D, 1)
    b2 = b2_ref[...]
    hs, ss = [], []
    for t in range(T):
        a = agg2_ref[0, t] + agg2_ref[1, t] + xs2_ref[t]
        ht = a * norm_ref[t][:, None] + b2
        st = jnp.dot(jnp.tanh(jnp.dot(ht, wa, preferred_element_type=jnp.float32,
                                      precision=_PREC)),
                     va, preferred_element_type=jnp.float32,
                     precision=_PREC)
        hs.append(ht)
        ss.append(st)
    m = jnp.maximum(jnp.maximum(ss[0], ss[1]), ss[2])
    es = [jnp.exp(s - m) for s in ss]
    den = es[0] + es[1] + es[2]
    h = (es[0] * hs[0] + es[1] * hs[1] + es[2] * hs[2]) / den
    hb = _bf16_rtne(h)
    p_ref[...] = jnp.dot(hb, _bf16_rtne(wtop_ref[...]),
                         preferred_element_type=jnp.float32)
    q_ref[...] = jnp.dot(hb, _bf16_rtne(wbot_ref[...]),
                         preferred_element_type=jnp.float32) + be1_ref[...]


def kernel(x0, x1, x2, edge_index0, edge_index1, edge_index2, edge_index_next,
           W1, b1, W2, b2, Wa, va, We1, be1, We2, be2):
    f32 = jnp.float32
    x = jnp.stack([x0, x1, x2])
    xpad = jnp.pad(x, ((0, 0), (0, NP - N), (0, 0)))
    src = jnp.concatenate([edge_index0[0], edge_index1[0], edge_index2[0]])
    dst = jnp.concatenate([edge_index0[1], edge_index1[1], edge_index2[1]])

    degp = _build_deg_kernel()(dst).reshape(NW, T, NP)

    xs, norm = pl.pallas_call(
        _b_body,
        grid=(NB,),
        in_specs=[
            pl.BlockSpec((T, BN, D), lambda i: (0, i, 0)),
            pl.BlockSpec((NW, T, BN), lambda i: (0, 0, i)),
            pl.BlockSpec((D, D), lambda i: (0, 0)),
        ],
        out_specs=[
            pl.BlockSpec((T, BN, D), lambda i: (0, i, 0)),
            pl.BlockSpec((T, BN), lambda i: (0, i)),
        ],
        out_shape=[
            jax.ShapeDtypeStruct((T, NP, D), f32),
            jax.ShapeDtypeStruct((T, NP), f32),
        ],
    )(xpad, degp, W1)

    agg = _build_segsum_kernel()(src, dst, xs.reshape(T * NP, D))

    xs2 = pl.pallas_call(
        _d_body,
        grid=(NB,),
        in_specs=[
            pl.BlockSpec((NC, T, BN, D), lambda i: (0, 0, i, 0)),
            pl.BlockSpec((T, BN, D), lambda i: (0, i, 0)),
            pl.BlockSpec((T, BN), lambda i: (0, i)),
            pl.BlockSpec((D,), lambda i: (0,)),
            pl.BlockSpec((D, D), lambda i: (0, 0)),
        ],
        out_specs=pl.BlockSpec((T, BN, D), lambda i: (0, i, 0)),
        out_shape=jax.ShapeDtypeStruct((T, NP, D), f32),
    )(agg, xs, norm, b1, W2)

    agg2 = _build_segsum_kernel()(src, dst, xs2.reshape(T * NP, D))

    P, Q = pl.pallas_call(
        _f_body,
        grid=(NB,),
        in_specs=[
            pl.BlockSpec((NC, T, BN, D), lambda i: (0, 0, i, 0)),
            pl.BlockSpec((T, BN, D), lambda i: (0, i, 0)),
            pl.BlockSpec((T, BN), lambda i: (0, i)),
            pl.BlockSpec((D,), lambda i: (0,)),
            pl.BlockSpec((D, D), lambda i: (0, 0)),
            pl.BlockSpec((D,), lambda i: (0,)),
            pl.BlockSpec((D, D), lambda i: (0, 0)),
            pl.BlockSpec((D, D), lambda i: (0, 0)),
            pl.BlockSpec((D,), lambda i: (0,)),
        ],
        out_specs=[
            pl.BlockSpec((BN, D), lambda i: (i, 0)),
            pl.BlockSpec((BN, D), lambda i: (i, 0)),
        ],
        out_shape=[
            jax.ShapeDtypeStruct((NP, D), f32),
            jax.ShapeDtypeStruct((NP, D), f32),
        ],
    )(agg2, xs2, norm, b2, Wa, va, We1[:D], We1[D:], be1)

    we2t = _bf16_rtne(We2)
    out = _build_edge_kernel()(P, Q, edge_index_next[0], edge_index_next[1],
                       we2t.reshape(D), jnp.broadcast_to(be2, (16,)))
    return out
